# baseline (speedup 1.0000x reference)
import jax
import jax.numpy as jnp
from jax.experimental import pallas as pl
from jax.experimental.pallas import tpu as pltpu

_N, _C = 32768, 100
_R = 4096


def _tc_body(x_ref, w_ref, b_ref, o_ref):
    t = x_ref[...] * w_ref[...]
    contrib = jnp.where(t != t, jnp.float32(0.0), t)
    o_ref[...] = jnp.sum(contrib, axis=1) + b_ref[0]


def kernel(local_map_predictions, weights_pool, bias):
    x = local_map_predictions
    w128 = jnp.zeros((1, 128), jnp.float32).at[0, :_C].set(weights_pool)
    out = pl.pallas_call(
        _tc_body,
        grid=(_N // _R,),
        in_specs=[
            pl.BlockSpec((_R, 128), lambda i: (i, 0)),
            pl.BlockSpec((1, 128), lambda i: (0, 0)),
            pl.BlockSpec(memory_space=pltpu.SMEM),
        ],
        out_specs=pl.BlockSpec((_R,), lambda i: (i,)),
        out_shape=jax.ShapeDtypeStruct((_N,), jnp.float32),
    )(x, w128, bias)
    return out[:, None]


# speedup vs baseline: 1.0346x; 1.0346x over previous
import jax
import jax.numpy as jnp
from jax.experimental import pallas as pl
from jax.experimental.pallas import tpu as pltpu

_N, _C = 32768, 100
_NCH = 16
_R = _N // _NCH


def _tc_body(x_hbm, w_ref, b_ref, o_ref, buf, sem):
    for i in range(_NCH):
        pltpu.make_async_copy(
            x_hbm.at[pl.ds(i * _R, _R), :], buf.at[i], sem.at[i]
        ).start()
    b = b_ref[0]
    w = w_ref[...]
    for i in range(_NCH):
        pltpu.make_async_copy(
            x_hbm.at[pl.ds(i * _R, _R), :], buf.at[i], sem.at[i]
        ).wait()
        t = buf[i] * w
        contrib = jnp.where(t != t, jnp.float32(0.0), t)
        o_ref[pl.ds(i * _R, _R)] = jnp.sum(contrib, axis=1) + b


def kernel(local_map_predictions, weights_pool, bias):
    x = local_map_predictions
    w2 = weights_pool[None, :]
    out = pl.pallas_call(
        _tc_body,
        in_specs=[
            pl.BlockSpec(memory_space=pl.ANY),
            pl.BlockSpec(memory_space=pltpu.VMEM),
            pl.BlockSpec(memory_space=pltpu.SMEM),
        ],
        out_specs=pl.BlockSpec(memory_space=pltpu.VMEM),
        out_shape=jax.ShapeDtypeStruct((_N,), jnp.float32),
        scratch_shapes=[
            pltpu.VMEM((_NCH, _R, _C), jnp.float32),
            pltpu.SemaphoreType.DMA((_NCH,)),
        ],
    )(x, w2, bias)
    return out[:, None]


# speedup vs baseline: 3.3482x; 3.2362x over previous
import jax
import jax.numpy as jnp
from jax.experimental import pallas as pl
from jax.experimental.pallas import tpu as pltpu

_N, _C = 32768, 100
_B = 4096


def _tc_body(xt_ref, w_ref, b_ref, o_ref):
    t = xt_ref[...] * w_ref[...]
    contrib = jnp.where(t != t, jnp.float32(0.0), t)
    o_ref[...] = jnp.sum(contrib, axis=0) + b_ref[0]


def kernel(local_map_predictions, weights_pool, bias):
    xt = jnp.swapaxes(local_map_predictions, 0, 1)
    w2 = weights_pool[:, None]
    out = pl.pallas_call(
        _tc_body,
        grid=(_N // _B,),
        in_specs=[
            pl.BlockSpec((_C, _B), lambda i: (0, i)),
            pl.BlockSpec((_C, 1), lambda i: (0, 0)),
            pl.BlockSpec(memory_space=pltpu.SMEM),
        ],
        out_specs=pl.BlockSpec((_B,), lambda i: (i,)),
        out_shape=jax.ShapeDtypeStruct((_N,), jnp.float32),
    )(xt, w2, bias)
    return out[:, None]


# speedup vs baseline: 3.4108x; 1.0187x over previous
import jax
import jax.numpy as jnp
from jax.experimental import pallas as pl
from jax.experimental.pallas import tpu as pltpu

_N, _C = 32768, 100
_S = 4
_B = 2048
_G = _N // (_S * _B)


def _tc_body(x0, x1, x2, x3, w_ref, b_ref, o_ref):
    w = w_ref[...]
    b = b_ref[0]
    for k, xr in enumerate((x0, x1, x2, x3)):
        t = xr[...] * w
        contrib = jnp.where(t != t, jnp.float32(0.0), t)
        o_ref[k, :] = jnp.sum(contrib, axis=0) + b


def _mk_spec(k):
    return pl.BlockSpec((_C, _B), lambda i, k=k: (0, k * _G + i))


def kernel(local_map_predictions, weights_pool, bias):
    xt = jnp.swapaxes(local_map_predictions, 0, 1)
    w2 = weights_pool[:, None]
    out = pl.pallas_call(
        _tc_body,
        grid=(_G,),
        in_specs=[
            _mk_spec(0),
            _mk_spec(1),
            _mk_spec(2),
            _mk_spec(3),
            pl.BlockSpec((_C, 1), lambda i: (0, 0)),
            pl.BlockSpec(memory_space=pltpu.SMEM),
        ],
        out_specs=pl.BlockSpec((_S, _B), lambda i: (0, i)),
        out_shape=jax.ShapeDtypeStruct((_S, _N // _S), jnp.float32),
    )(xt, xt, xt, xt, w2, bias)
    return out.reshape(_N)[:, None]


# speedup vs baseline: 3.4850x; 1.0218x over previous
import jax
import jax.numpy as jnp
from jax.experimental import pallas as pl
from jax.experimental.pallas import tpu as pltpu

_N, _C = 32768, 100
_S = 4
_B = 4096
_G = _N // (_S * _B)


def _tc_body(x0, x1, x2, x3, w_ref, b_ref, o_ref):
    w = w_ref[...]
    b = b_ref[0]
    for k, xr in enumerate((x0, x1, x2, x3)):
        t = xr[...] * w
        contrib = jnp.where(t != t, jnp.float32(0.0), t)
        o_ref[k, :] = jnp.sum(contrib, axis=0) + b


def _mk_spec(k):
    return pl.BlockSpec((_C, _B), lambda i, k=k: (0, k * _G + i))


def kernel(local_map_predictions, weights_pool, bias):
    xt = jnp.swapaxes(local_map_predictions, 0, 1)
    w2 = weights_pool[:, None]
    out = pl.pallas_call(
        _tc_body,
        grid=(_G,),
        in_specs=[
            _mk_spec(0),
            _mk_spec(1),
            _mk_spec(2),
            _mk_spec(3),
            pl.BlockSpec((_C, 1), lambda i: (0, 0)),
            pl.BlockSpec(memory_space=pltpu.SMEM),
        ],
        out_specs=pl.BlockSpec((_S, _B), lambda i: (0, i)),
        out_shape=jax.ShapeDtypeStruct((_S, _N // _S), jnp.float32),
    )(xt, xt, xt, xt, w2, bias)
    return out.reshape(_N)[:, None]
